# Initial kernel scaffold; baseline (speedup 1.0000x reference)
import functools

import jax
import jax.numpy as jnp
from jax import lax
from jax.experimental import pallas as pl
from jax.experimental.pallas import tpu as pltpu
from jax.experimental.pallas import tpu_sc as plsc

EPS = 1e-8
NC, NS = 2, 16
NW = NC * NS
CHUNK = 128
MSG_W = 192
CNT_COL = 176

_f32 = jnp.float32


def _dot(a, b):
    return jnp.dot(a, b, preferred_element_type=_f32)


def _vnorm_sq(v3):
    vx, vy, vz = v3
    return jnp.clip(vx * vx + vy * vy + vz * vz, EPS, None)


def _vec_gate(v3):
    g = jax.nn.sigmoid(jnp.sqrt(_vnorm_sq(v3)))
    return [c * g for c in v3]


def _layernorm_sv(w, b, s, v3):
    mu = jnp.mean(s, axis=1, keepdims=True)
    var = jnp.mean((s - mu) * (s - mu), axis=1, keepdims=True)
    s = (s - mu) / jnp.sqrt(var + 1e-5) * w + b
    denom = jnp.sqrt(jnp.mean(_vnorm_sq(v3), axis=1, keepdims=True))
    return s, [c / denom for c in v3]


def _prep_input_wts(p):
    return (p['wh'].T, p['ws_w'][:, :6].T, p['ws_w'][:, 6:].T,
            p['ws_b'][None, :], p['wv'].T)


def _prep_edge_wts(c):
    p0, p1, p2 = c['msg0'], c['msg1'], c['msg2']
    wh, W = p0['wh'], p0['ws_w']
    return (wh[:, :16].T, wh[:, 16:17].T, wh[:, 17:].T,
            W[:, :128].T, W[:, 128:160].T, W[:, 160:288].T, W[:, 288:].T,
            p0['ws_b'][None, :], p0['wv'].T,
            p1['wh'].T, p1['ws_w'][:, :128].T, p1['ws_w'][:, 128:].T,
            p1['ws_b'][None, :], p1['wv'].T,
            p2['wh'].T, p2['ws_w'][:, :128].T, p2['ws_w'][:, 128:].T,
            p2['ws_b'][None, :], p2['wv'].T)


def _prep_node_wts(lp):
    f0, f1 = lp['ff0'], lp['ff1']
    return (lp['norm0']['w'][None, :], lp['norm0']['b'][None, :],
            f0['wh'].T, f0['ws_w'][:, :128].T, f0['ws_w'][:, 128:].T,
            f0['ws_b'][None, :], f0['wv'].T,
            f1['wh'].T, f1['ws_w'][:, :512].T, f1['ws_w'][:, 512:].T,
            f1['ws_b'][None, :], f1['wv'].T,
            lp['norm1']['w'][None, :], lp['norm1']['b'][None, :])


def _input_math(xs, xv3, wts):
    whT, wsAT, wsBT, b, wvT = wts
    vh = [_dot(c, whT) for c in xv3]
    vn = jnp.sqrt(_vnorm_sq(vh))
    s = jax.nn.relu(_dot(xs, wsAT) + _dot(vn, wsBT) + b)
    v = _vec_gate([_dot(c, wvT) for c in vh])
    return s, v


def _edge_math(sS, sD, eS, vS3, vD3, eV3, wts):
    (whAT, whB, whCT, W1T, W2T, W3T, W4T, b0, wv0T,
     wh1T, wsA1T, wsB1T, b1, wv1T,
     wh2T, wsA2T, wsB2T, b2, wv2T) = wts
    vh = [_dot(vS3[k], whAT) + eV3[k] * whB + _dot(vD3[k], whCT)
          for k in range(3)]
    vn = jnp.sqrt(_vnorm_sq(vh))
    s = _dot(sS, W1T) + _dot(eS, W2T) + _dot(sD, W3T) + _dot(vn, W4T) + b0
    s = jax.nn.relu(s)
    v = _vec_gate([_dot(c, wv0T) for c in vh])
    vh = [_dot(c, wh1T) for c in v]
    vn = jnp.sqrt(_vnorm_sq(vh))
    s = jax.nn.relu(_dot(s, wsA1T) + _dot(vn, wsB1T) + b1)
    v = _vec_gate([_dot(c, wv1T) for c in vh])
    vh = [_dot(c, wh2T) for c in v]
    vn = jnp.sqrt(_vnorm_sq(vh))
    s = _dot(s, wsA2T) + _dot(vn, wsB2T) + b2
    v = [_dot(c, wv2T) for c in vh]
    return s, v


def _node_math(s, v3, agg, wts, fin):
    (n0w, n0b, wh0T, wsA0T, wsB0T, b0, wv0T,
     wh1T, wsA1T, wsB1T, b1, wv1T, n1w, n1b) = wts
    cnt = jnp.maximum(agg[:, CNT_COL:CNT_COL + 1], 1.0)
    s = s + agg[:, :128] / cnt
    v3 = [v3[k] + agg[:, 128 + 16 * k:144 + 16 * k] / cnt for k in range(3)]
    s, v3 = _layernorm_sv(n0w, n0b, s, v3)
    vh = [_dot(c, wh0T) for c in v3]
    vn = jnp.sqrt(_vnorm_sq(vh))
    fs = jax.nn.relu(_dot(s, wsA0T) + _dot(vn, wsB0T) + b0)
    fv = _vec_gate([_dot(c, wv0T) for c in vh])
    vh = [_dot(c, wh1T) for c in fv]
    vn = jnp.sqrt(_vnorm_sq(vh))
    fs = _dot(fs, wsA1T) + _dot(vn, wsB1T) + b1
    fv = [_dot(c, wv1T) for c in vh]
    s, v3 = _layernorm_sv(n1w, n1b, s + fs, [v3[k] + fv[k] for k in range(3)])
    if fin is not None:
        s, v3 = _layernorm_sv(fin[0], fin[1], s, v3)
    return s, v3


def _full_spec(a):
    nd = a.ndim
    return pl.BlockSpec(a.shape, lambda i, _nd=nd: (0,) * _nd)


def _row_spec(nrows, ncols):
    return pl.BlockSpec((nrows, ncols), lambda i: (i, 0))


def _call_input_tc(xs, xv9, wts):
    N = xs.shape[0]
    B = 2000
    nw = len(wts)

    def body(*refs):
        xs_r, xv_r = refs[0], refs[1]
        w = [r[...] for r in refs[2:2 + nw]]
        s_o, v_o = refs[2 + nw], refs[3 + nw]
        xv3 = [xv_r[:, 3 * k:3 * k + 3] for k in range(3)]
        s, v = _input_math(xs_r[...], xv3, w)
        s_o[...] = s
        v_o[...] = jnp.concatenate(v, axis=1)

    return pl.pallas_call(
        body,
        grid=(N // B,),
        in_specs=[_row_spec(B, 6), _row_spec(B, 9)] + [_full_spec(w) for w in wts],
        out_specs=[_row_spec(B, 128), _row_spec(B, 48)],
        out_shape=[jax.ShapeDtypeStruct((N, 128), _f32),
                   jax.ShapeDtypeStruct((N, 48), _f32)],
    )(xs, xv9, *wts)


def _call_edge_tc(sS, sD, eS, vS, vD, eV3, wts):
    E = sS.shape[0]
    B = 640
    nw = len(wts)

    def body(*refs):
        sS_r, sD_r, eS_r, vS_r, vD_r, eV_r = refs[:6]
        w = [r[...] for r in refs[6:6 + nw]]
        out = refs[6 + nw]
        vS3 = [vS_r[:, 16 * k:16 * k + 16] for k in range(3)]
        vD3 = [vD_r[:, 16 * k:16 * k + 16] for k in range(3)]
        eV = [eV_r[:, k:k + 1] for k in range(3)]
        s, v = _edge_math(sS_r[...], sD_r[...], eS_r[...], vS3, vD3, eV, w)
        ones = jnp.ones((s.shape[0], 1), _f32)
        pad = jnp.zeros((s.shape[0], MSG_W - CNT_COL - 1), _f32)
        out[...] = jnp.concatenate([s] + v + [ones, pad], axis=1)

    return pl.pallas_call(
        body,
        grid=(E // B,),
        in_specs=[_row_spec(B, 128), _row_spec(B, 128), _row_spec(B, 32),
                  _row_spec(B, 48), _row_spec(B, 48), _row_spec(B, 3)]
                 + [_full_spec(w) for w in wts],
        out_specs=[_row_spec(B, MSG_W)],
        out_shape=[jax.ShapeDtypeStruct((E, MSG_W), _f32)],
    )(sS, sD, eS, vS, vD, eV3, *wts)[0]


def _call_node_tc(s, v48, parts, wts, fin):
    N = s.shape[0]
    B = 2000
    nw = len(wts)
    fwts = list(fin) if fin is not None else []
    nf = len(fwts)

    def body(*refs):
        s_r, v_r, p_r = refs[:3]
        w = [r[...] for r in refs[3:3 + nw]]
        f = [r[...] for r in refs[3 + nw:3 + nw + nf]]
        s_o, v_o = refs[3 + nw + nf], refs[4 + nw + nf]
        v3 = [v_r[:, 16 * k:16 * k + 16] for k in range(3)]
        agg = p_r[0] + p_r[1]
        so, vo = _node_math(s_r[...], v3, agg, w, f if nf else None)
        s_o[...] = so
        v_o[...] = jnp.concatenate(vo, axis=1)

    return pl.pallas_call(
        body,
        grid=(N // B,),
        in_specs=[_row_spec(B, 128), _row_spec(B, 48),
                  pl.BlockSpec((2, B, MSG_W), lambda i: (0, i, 0))]
                 + [_full_spec(w) for w in list(wts) + fwts],
        out_specs=[_row_spec(B, 128), _row_spec(B, 48)],
        out_shape=[jax.ShapeDtypeStruct((N, 128), _f32),
                   jax.ShapeDtypeStruct((N, 48), _f32)],
    )(s, v48, parts, *wts, *fwts)


def _sc_mesh():
    return plsc.VectorSubcoreMesh(core_axis_name="c", subcore_axis_name="s",
                                  num_cores=NC, num_subcores=NS)


def _sc_gather(s_tab, v_tab, src, dst):
    E = src.shape[0]
    nfull = E // (NW * CHUNK)
    rem_chunks = (E - nfull * NW * CHUNK) // CHUNK

    @functools.partial(
        pl.kernel,
        out_type=(jax.ShapeDtypeStruct((E, 128), _f32),
                  jax.ShapeDtypeStruct((E, 128), _f32),
                  jax.ShapeDtypeStruct((E, 48), _f32),
                  jax.ShapeDtypeStruct((E, 48), _f32)),
        mesh=_sc_mesh(),
        scratch_types=[pltpu.VMEM((CHUNK,), jnp.int32),
                       pltpu.VMEM((CHUNK,), jnp.int32),
                       pltpu.VMEM((CHUNK, 128), _f32),
                       pltpu.VMEM((CHUNK, 128), _f32),
                       pltpu.VMEM((CHUNK, 48), _f32),
                       pltpu.VMEM((CHUNK, 48), _f32),
                       pltpu.SemaphoreType.DMA],
    )
    def gather_k(s_hbm, v_hbm, src_hbm, dst_hbm, sS_o, sD_o, vS_o, vD_o,
                 idx_s, idx_d, buf_ss, buf_sd, buf_vs, buf_vd, sem):
        wid = lax.axis_index("s") * NC + lax.axis_index("c")

        def do_chunk(base):
            pltpu.sync_copy(src_hbm.at[pl.ds(base, CHUNK)], idx_s)
            pltpu.sync_copy(dst_hbm.at[pl.ds(base, CHUNK)], idx_d)
            d0 = pltpu.async_copy(s_hbm.at[idx_s], buf_ss, sem)
            d1 = pltpu.async_copy(s_hbm.at[idx_d], buf_sd, sem)
            d2 = pltpu.async_copy(v_hbm.at[idx_s], buf_vs, sem)
            d3 = pltpu.async_copy(v_hbm.at[idx_d], buf_vd, sem)
            d0.wait(); d1.wait(); d2.wait(); d3.wait()
            pltpu.sync_copy(buf_ss, sS_o.at[pl.ds(base, CHUNK)])
            pltpu.sync_copy(buf_sd, sD_o.at[pl.ds(base, CHUNK)])
            pltpu.sync_copy(buf_vs, vS_o.at[pl.ds(base, CHUNK)])
            pltpu.sync_copy(buf_vd, vD_o.at[pl.ds(base, CHUNK)])

        def loop_body(j, carry):
            do_chunk((j * NW + wid) * CHUNK)
            return carry

        lax.fori_loop(0, nfull, loop_body, 0)
        if rem_chunks:
            @pl.when(wid < rem_chunks)
            def _():
                do_chunk((nfull * NW + wid) * CHUNK)

    return gather_k(s_tab, v_tab, src, dst)


def _sc_scatter(msg, dst, zeros_blk, n_nodes):
    E = dst.shape[0]
    nfull = E // (NW * CHUNK)
    rem_chunks = (E - nfull * NW * CHUNK) // CHUNK
    rows_per_sub = n_nodes // NS

    @functools.partial(
        pl.kernel,
        out_type=jax.ShapeDtypeStruct((NC, n_nodes, MSG_W), _f32),
        mesh=_sc_mesh(),
        scratch_types=[pltpu.VMEM((CHUNK,), jnp.int32),
                       pltpu.VMEM((CHUNK, MSG_W), _f32),
                       pltpu.VMEM_SHARED((n_nodes, MSG_W), _f32),
                       pltpu.SemaphoreType.DMA],
    )
    def scatter_k(msg_hbm, dst_hbm, zero_hbm, out_hbm, idx_v, buf, acc, sem):
        cid = lax.axis_index("c")
        sid = lax.axis_index("s")
        wid = sid * NC + cid
        pltpu.sync_copy(zero_hbm, acc.at[pl.ds(sid * rows_per_sub, rows_per_sub)])
        plsc.subcore_barrier()

        def do_chunk(base):
            pltpu.sync_copy(dst_hbm.at[pl.ds(base, CHUNK)], idx_v)
            pltpu.sync_copy(msg_hbm.at[pl.ds(base, CHUNK)], buf)
            pltpu.sync_copy(buf, acc.at[idx_v], add=True)

        def loop_body(j, carry):
            do_chunk((j * NW + wid) * CHUNK)
            return carry

        lax.fori_loop(0, nfull, loop_body, 0)
        if rem_chunks:
            @pl.when(wid < rem_chunks)
            def _():
                do_chunk((nfull * NW + wid) * CHUNK)
        plsc.subcore_barrier()
        pltpu.sync_copy(acc.at[pl.ds(sid * rows_per_sub, rows_per_sub)],
                        out_hbm.at[cid, pl.ds(sid * rows_per_sub, rows_per_sub)])

    return scatter_k(msg, dst, zeros_blk)


def kernel(x_s, x_v, edge_index, edge_s, edge_v, params):
    N = x_s.shape[0]
    src, dst = edge_index[0], edge_index[1]
    xv9 = jnp.transpose(x_v, (0, 2, 1)).reshape(N, 9)
    ev3 = edge_v[:, 0, :]
    zeros_blk = jnp.zeros((N // NS, MSG_W), _f32)

    s, v48 = _call_input_tc(x_s, xv9, _prep_input_wts(params['input_proj']))
    n_layers = len(params['layers'])
    for li, lp in enumerate(params['layers']):
        sS, sD, vS, vD = _sc_gather(s, v48, src, dst)
        msg = _call_edge_tc(sS, sD, edge_s, vS, vD, ev3,
                            _prep_edge_wts(lp['conv']))
        parts = _sc_scatter(msg, dst, zeros_blk, N)
        fin = None
        if li == n_layers - 1:
            fn = params['final_norm']
            fin = (fn['w'][None, :], fn['b'][None, :])
        s, v48 = _call_node_tc(s, v48, parts, _prep_node_wts(lp), fin)

    v = jnp.stack([v48[:, :16], v48[:, 16:32], v48[:, 32:48]], axis=-1)
    return s, v


# speedup vs baseline: 11.2747x; 11.2747x over previous
import functools

import jax
import jax.numpy as jnp
from jax import lax
from jax.experimental import pallas as pl
from jax.experimental.pallas import tpu as pltpu
from jax.experimental.pallas import tpu_sc as plsc

EPS = 1e-8
NC, NS = 2, 16
NW = NC * NS
CHUNK = 128
PKW = 256
CNT_COL = 48

_f32 = jnp.float32


def _dot(a, b):
    return jnp.dot(a, b, preferred_element_type=_f32)


def _vnorm_sq(v3):
    vx, vy, vz = v3
    return jnp.clip(vx * vx + vy * vy + vz * vz, EPS, None)


def _vec_gate(v3):
    g = jax.nn.sigmoid(jnp.sqrt(_vnorm_sq(v3)))
    return [c * g for c in v3]


def _layernorm_sv(w, b, s, v3):
    mu = jnp.mean(s, axis=1, keepdims=True)
    var = jnp.mean((s - mu) * (s - mu), axis=1, keepdims=True)
    s = (s - mu) / jnp.sqrt(var + 1e-5) * w + b
    denom = jnp.sqrt(jnp.mean(_vnorm_sq(v3), axis=1, keepdims=True))
    return s, [c / denom for c in v3]


def _prep_input_wts(p):
    return (p['wh'].T, p['ws_w'][:, :6].T, p['ws_w'][:, 6:].T,
            p['ws_b'][None, :], p['wv'].T)


def _prep_edge_wts(c):
    p0, p1, p2 = c['msg0'], c['msg1'], c['msg2']
    wh, W = p0['wh'], p0['ws_w']
    return (wh[:, :16].T, wh[:, 16:17].T, wh[:, 17:].T,
            W[:, :128].T, W[:, 128:160].T, W[:, 160:288].T, W[:, 288:].T,
            p0['ws_b'][None, :], p0['wv'].T,
            p1['wh'].T, p1['ws_w'][:, :128].T, p1['ws_w'][:, 128:].T,
            p1['ws_b'][None, :], p1['wv'].T,
            p2['wh'].T, p2['ws_w'][:, :128].T, p2['ws_w'][:, 128:].T,
            p2['ws_b'][None, :], p2['wv'].T)


def _prep_node_wts(lp):
    f0, f1 = lp['ff0'], lp['ff1']
    return (lp['norm0']['w'][None, :], lp['norm0']['b'][None, :],
            f0['wh'].T, f0['ws_w'][:, :128].T, f0['ws_w'][:, 128:].T,
            f0['ws_b'][None, :], f0['wv'].T,
            f1['wh'].T, f1['ws_w'][:, :512].T, f1['ws_w'][:, 512:].T,
            f1['ws_b'][None, :], f1['wv'].T,
            lp['norm1']['w'][None, :], lp['norm1']['b'][None, :])


def _input_math(xs, xv3, wts):
    whT, wsAT, wsBT, b, wvT = wts
    vh = [_dot(c, whT) for c in xv3]
    vn = jnp.sqrt(_vnorm_sq(vh))
    s = jax.nn.relu(_dot(xs, wsAT) + _dot(vn, wsBT) + b)
    v = _vec_gate([_dot(c, wvT) for c in vh])
    return s, v


def _edge_math(sS, sD, eS, vS3, vD3, eV3, wts):
    (whAT, whB, whCT, W1T, W2T, W3T, W4T, b0, wv0T,
     wh1T, wsA1T, wsB1T, b1, wv1T,
     wh2T, wsA2T, wsB2T, b2, wv2T) = wts
    vh = [_dot(vS3[k], whAT) + eV3[k] * whB + _dot(vD3[k], whCT)
          for k in range(3)]
    vn = jnp.sqrt(_vnorm_sq(vh))
    s = _dot(sS, W1T) + _dot(eS, W2T) + _dot(sD, W3T) + _dot(vn, W4T) + b0
    s = jax.nn.relu(s)
    v = _vec_gate([_dot(c, wv0T) for c in vh])
    vh = [_dot(c, wh1T) for c in v]
    vn = jnp.sqrt(_vnorm_sq(vh))
    s = jax.nn.relu(_dot(s, wsA1T) + _dot(vn, wsB1T) + b1)
    v = _vec_gate([_dot(c, wv1T) for c in vh])
    vh = [_dot(c, wh2T) for c in v]
    vn = jnp.sqrt(_vnorm_sq(vh))
    s = _dot(s, wsA2T) + _dot(vn, wsB2T) + b2
    v = [_dot(c, wv2T) for c in vh]
    return s, v


def _node_math(s, v3, agg_s, agg_v, wts, fin):
    (n0w, n0b, wh0T, wsA0T, wsB0T, b0, wv0T,
     wh1T, wsA1T, wsB1T, b1, wv1T, n1w, n1b) = wts
    cnt = jnp.maximum(agg_v[:, CNT_COL:CNT_COL + 1], 1.0)
    s = s + agg_s / cnt
    v3 = [v3[k] + agg_v[:, 16 * k:16 * k + 16] / cnt for k in range(3)]
    s, v3 = _layernorm_sv(n0w, n0b, s, v3)
    vh = [_dot(c, wh0T) for c in v3]
    vn = jnp.sqrt(_vnorm_sq(vh))
    fs = jax.nn.relu(_dot(s, wsA0T) + _dot(vn, wsB0T) + b0)
    fv = _vec_gate([_dot(c, wv0T) for c in vh])
    vh = [_dot(c, wh1T) for c in fv]
    vn = jnp.sqrt(_vnorm_sq(vh))
    fs = _dot(fs, wsA1T) + _dot(vn, wsB1T) + b1
    fv = [_dot(c, wv1T) for c in vh]
    s, v3 = _layernorm_sv(n1w, n1b, s + fs, [v3[k] + fv[k] for k in range(3)])
    if fin is not None:
        s, v3 = _layernorm_sv(fin[0], fin[1], s, v3)
    return s, v3


def _full_spec(a):
    nd = a.ndim
    return pl.BlockSpec(a.shape, lambda i, _nd=nd: (0,) * _nd)


def _row_spec(nrows, ncols):
    return pl.BlockSpec((nrows, ncols), lambda i: (i, 0))


def _pack(s, v):
    pad = jnp.zeros((s.shape[0], PKW - 176), _f32)
    return jnp.concatenate([s] + v + [pad], axis=1)


def _call_input_tc(xs, xv9, wts):
    N = xs.shape[0]
    B = 2000
    nw = len(wts)

    def body(*refs):
        xs_r, xv_r = refs[0], refs[1]
        w = [r[...] for r in refs[2:2 + nw]]
        p_o = refs[2 + nw]
        xv3 = [xv_r[:, 3 * k:3 * k + 3] for k in range(3)]
        s, v = _input_math(xs_r[...], xv3, w)
        p_o[...] = _pack(s, v)

    return pl.pallas_call(
        body,
        grid=(N // B,),
        in_specs=[_row_spec(B, 6), _row_spec(B, 9)] + [_full_spec(w) for w in wts],
        out_specs=[_row_spec(B, PKW)],
        out_shape=[jax.ShapeDtypeStruct((N, PKW), _f32)],
    )(xs, xv9, *wts)[0]


def _call_edge_tc(A, B_, eS, eV3, wts):
    E = A.shape[0]
    B = 640
    nw = len(wts)

    def body(*refs):
        A_r, B_r, eS_r, eV_r = refs[:4]
        w = [r[...] for r in refs[4:4 + nw]]
        ms_o, mv_o = refs[4 + nw], refs[5 + nw]
        sS = A_r[:, :128]
        sD = B_r[:, :128]
        vS3 = [A_r[:, 128 + 16 * k:144 + 16 * k] for k in range(3)]
        vD3 = [B_r[:, 128 + 16 * k:144 + 16 * k] for k in range(3)]
        eV = [eV_r[:, k:k + 1] for k in range(3)]
        s, v = _edge_math(sS, sD, eS_r[...], vS3, vD3, eV, w)
        ones = jnp.ones((s.shape[0], 1), _f32)
        pad = jnp.zeros((s.shape[0], 128 - CNT_COL - 1), _f32)
        ms_o[...] = s
        mv_o[...] = jnp.concatenate(v + [ones, pad], axis=1)

    return pl.pallas_call(
        body,
        grid=(E // B,),
        in_specs=[_row_spec(B, PKW), _row_spec(B, PKW), _row_spec(B, 32),
                  _row_spec(B, 3)]
                 + [_full_spec(w) for w in wts],
        out_specs=[_row_spec(B, 128), _row_spec(B, 128)],
        out_shape=[jax.ShapeDtypeStruct((E, 128), _f32),
                   jax.ShapeDtypeStruct((E, 128), _f32)],
    )(A, B_, eS, eV3, *wts)


def _call_node_tc(pk, parts_s, parts_v, wts, fin):
    N = pk.shape[0]
    B = 2000
    nw = len(wts)
    fwts = list(fin) if fin is not None else []
    nf = len(fwts)

    def body(*refs):
        p_r, ps_r, pv_r = refs[:3]
        w = [r[...] for r in refs[3:3 + nw]]
        f = [r[...] for r in refs[3 + nw:3 + nw + nf]]
        p_o = refs[3 + nw + nf]
        s = p_r[:, :128]
        v3 = [p_r[:, 128 + 16 * k:144 + 16 * k] for k in range(3)]
        agg_s = ps_r[0] + ps_r[1]
        agg_v = pv_r[0] + pv_r[1]
        so, vo = _node_math(s, v3, agg_s, agg_v, w, f if nf else None)
        p_o[...] = _pack(so, vo)

    return pl.pallas_call(
        body,
        grid=(N // B,),
        in_specs=[_row_spec(B, PKW),
                  pl.BlockSpec((2, B, 128), lambda i: (0, i, 0)),
                  pl.BlockSpec((2, B, 128), lambda i: (0, i, 0))]
                 + [_full_spec(w) for w in list(wts) + fwts],
        out_specs=[_row_spec(B, PKW)],
        out_shape=[jax.ShapeDtypeStruct((N, PKW), _f32)],
    )(pk, parts_s, parts_v, *wts, *fwts)[0]


def _sc_mesh():
    return plsc.VectorSubcoreMesh(core_axis_name="c", subcore_axis_name="s",
                                  num_cores=NC, num_subcores=NS)


def _sub_rows(n_nodes, sid):
    per = (n_nodes // NS) // 8 * 8
    last = n_nodes - (NS - 1) * per
    return per, last


def _sc_gather(pk_tab, src, dst):
    E = src.shape[0]
    nfull = E // (NW * CHUNK)
    rem_chunks = (E - nfull * NW * CHUNK) // CHUNK

    @functools.partial(
        pl.kernel,
        out_type=(jax.ShapeDtypeStruct((E, PKW), _f32),
                  jax.ShapeDtypeStruct((E, PKW), _f32)),
        mesh=_sc_mesh(),
        scratch_types=[pltpu.VMEM((CHUNK,), jnp.int32),
                       pltpu.VMEM((CHUNK,), jnp.int32),
                       pltpu.VMEM((CHUNK, PKW), _f32),
                       pltpu.VMEM((CHUNK, PKW), _f32),
                       pltpu.SemaphoreType.DMA],
    )
    def gather_k(pk_hbm, src_hbm, dst_hbm, A_o, B_o,
                 idx_s, idx_d, buf_a, buf_b, sem):
        wid = lax.axis_index("s") * NC + lax.axis_index("c")

        def do_chunk(base):
            pltpu.sync_copy(src_hbm.at[pl.ds(base, CHUNK)], idx_s)
            pltpu.sync_copy(dst_hbm.at[pl.ds(base, CHUNK)], idx_d)
            d0 = pltpu.async_copy(pk_hbm.at[idx_s], buf_a, sem)
            d1 = pltpu.async_copy(pk_hbm.at[idx_d], buf_b, sem)
            d0.wait(); d1.wait()
            pltpu.sync_copy(buf_a, A_o.at[pl.ds(base, CHUNK)])
            pltpu.sync_copy(buf_b, B_o.at[pl.ds(base, CHUNK)])

        def loop_body(j, carry):
            do_chunk((j * NW + wid) * CHUNK)
            return carry

        lax.fori_loop(0, nfull, loop_body, 0)
        if rem_chunks:
            @pl.when(wid < rem_chunks)
            def _():
                do_chunk((nfull * NW + wid) * CHUNK)

    return gather_k(pk_tab, src, dst)


def _sc_scatter2(ms, mv, dst, zeros_blk, n_nodes):
    E = dst.shape[0]
    nfull = E // (NW * CHUNK)
    rem_chunks = (E - nfull * NW * CHUNK) // CHUNK
    per, last = _sub_rows(n_nodes, None)

    @functools.partial(
        pl.kernel,
        out_type=(jax.ShapeDtypeStruct((NC, n_nodes, 128), _f32),
                  jax.ShapeDtypeStruct((NC, n_nodes, 128), _f32)),
        mesh=_sc_mesh(),
        scratch_types=[pltpu.VMEM((CHUNK,), jnp.int32),
                       pltpu.VMEM((CHUNK, 128), _f32),
                       pltpu.VMEM_SHARED((n_nodes, 128), _f32),
                       pltpu.SemaphoreType.DMA],
    )
    def scatter_k(ms_hbm, mv_hbm, dst_hbm, zero_hbm, outs_hbm, outv_hbm,
                  idx_v, buf, acc, sem):
        cid = lax.axis_index("c")
        sid = lax.axis_index("s")
        wid = sid * NC + cid

        def my_rows(fn):
            @pl.when(sid < NS - 1)
            def _():
                fn(sid * per, per)

            @pl.when(sid == NS - 1)
            def _():
                fn(sid * per, last)

        def zero_own(lo, n):
            pltpu.sync_copy(zero_hbm.at[pl.ds(0, n)], acc.at[pl.ds(lo, n)])

        def scatter_pass(msg_hbm):
            def do_chunk(base):
                pltpu.sync_copy(dst_hbm.at[pl.ds(base, CHUNK)], idx_v)
                pltpu.sync_copy(msg_hbm.at[pl.ds(base, CHUNK)], buf)
                pltpu.sync_copy(buf, acc.at[idx_v], add=True)

            def loop_body(j, carry):
                do_chunk((j * NW + wid) * CHUNK)
                return carry

            lax.fori_loop(0, nfull, loop_body, 0)
            if rem_chunks:
                @pl.when(wid < rem_chunks)
                def _():
                    do_chunk((nfull * NW + wid) * CHUNK)

        my_rows(zero_own)
        plsc.subcore_barrier()
        scatter_pass(ms_hbm)
        plsc.subcore_barrier()

        def drain_s(lo, n):
            pltpu.sync_copy(acc.at[pl.ds(lo, n)], outs_hbm.at[cid, pl.ds(lo, n)])
            zero_own(lo, n)

        my_rows(drain_s)
        plsc.subcore_barrier()
        scatter_pass(mv_hbm)
        plsc.subcore_barrier()

        def drain_v(lo, n):
            pltpu.sync_copy(acc.at[pl.ds(lo, n)], outv_hbm.at[cid, pl.ds(lo, n)])

        my_rows(drain_v)

    return scatter_k(ms, mv, dst, zeros_blk)


def kernel(x_s, x_v, edge_index, edge_s, edge_v, params):
    N = x_s.shape[0]
    src, dst = edge_index[0], edge_index[1]
    xv9 = jnp.transpose(x_v, (0, 2, 1)).reshape(N, 9)
    ev3 = edge_v[:, 0, :]
    _, zlast = _sub_rows(N, None)
    zeros_blk = jnp.zeros((zlast, 128), _f32)

    pk = _call_input_tc(x_s, xv9, _prep_input_wts(params['input_proj']))
    n_layers = len(params['layers'])
    for li, lp in enumerate(params['layers']):
        A, B_ = _sc_gather(pk, src, dst)
        ms, mv = _call_edge_tc(A, B_, edge_s, ev3, _prep_edge_wts(lp['conv']))
        parts_s, parts_v = _sc_scatter2(ms, mv, dst, zeros_blk, N)
        fin = None
        if li == n_layers - 1:
            fn = params['final_norm']
            fin = (fn['w'][None, :], fn['b'][None, :])
        pk = _call_node_tc(pk, parts_s, parts_v, _prep_node_wts(lp), fin)

    s = pk[:, :128]
    v = jnp.stack([pk[:, 128:144], pk[:, 144:160], pk[:, 160:176]], axis=-1)
    return s, v
